# Initial kernel scaffold; baseline (speedup 1.0000x reference)
import jax
import jax.numpy as jnp
from jax.experimental import pallas as pl


def kernel(cam_feats, rots, trans, intrins, post_rots, post_trans):
    raise NotImplementedError("write your pallas kernel here")


# speedup vs baseline: 1.6766x; 1.6766x over previous
import functools

import jax
import jax.numpy as jnp
from jax import lax
from jax.experimental import pallas as pl
from jax.experimental.pallas import tpu as pltpu
from jax.experimental.pallas import tpu_sc as plsc

B, N, D, fH, fW, C = 4, 6, 41, 8, 22, 64
ogfH, ogfW = 128, 352
NX0, NX1, NZ = 200, 200, 1
HW = fH * fW
P_CAM = D * HW
NCAM = B * N
NPRIME = NCAM * P_CAM
P_BATCH = NPRIME // B
NVOX = NX0 * NX1

NCORE, NTILE = 2, 16
CH = C // NCORE
PT_TILE = 2704
PT_LAST = P_BATCH - (NTILE - 1) * PT_TILE
ROWS_TILE = 22
PT_PAD = ROWS_TILE * 128
FROWS = 10
FPTS = FROWS * 128
ACC_ROWS = 40016
ZROWS = 128
ZPT = 2501
DPT = NVOX // NTILE


def _idx_body(px_ref, py_ref, pz_ref, idx_ref):
    gx = ((px_ref[...] - (-50.0)) / 0.5).astype(jnp.int32)
    gy = ((py_ref[...] - (-50.0)) / 0.5).astype(jnp.int32)
    gz = ((pz_ref[...] - (-10.0)) / 20.0).astype(jnp.int32)
    kept = ((gx >= 0) & (gx < NX0) & (gy >= 0) & (gy < NX1)
            & (gz >= 0) & (gz < NZ))
    idx_ref[...] = jnp.where(kept, gx * NX1 + gy, NVOX)


def _point_indices(points):
    px = points[..., 0]
    py = points[..., 1]
    pz = points[..., 2]
    return pl.pallas_call(
        _idx_body,
        out_shape=jax.ShapeDtypeStruct((NCAM, D, HW), jnp.int32),
    )(px, py, pz)


def _splat_body(feats, idxp, out, fbuf, ibuf, zbuf, acc):
    cid = lax.axis_index("c")
    tid = lax.axis_index("s")
    c0 = cid * CH

    zeros16 = jnp.zeros((16,), jnp.float32)

    @pl.loop(0, ZROWS)
    def _(r):
        zbuf[r, pl.ds(0, 16)] = zeros16
        zbuf[r, pl.ds(16, 16)] = zeros16

    @pl.loop(0, B)
    def _(b):
        zbase = tid * ZPT

        @pl.loop(0, ZPT // ZROWS)
        def _(k):
            pltpu.sync_copy(zbuf, acc.at[pl.ds(zbase + k * ZROWS, ZROWS)])

        pltpu.sync_copy(zbuf.at[pl.ds(0, ZPT % ZROWS)],
                        acc.at[pl.ds(zbase + (ZPT // ZROWS) * ZROWS,
                                     ZPT % ZROWS)])
        plsc.subcore_barrier()

        pbase = b * P_BATCH + tid * PT_TILE
        pltpu.sync_copy(idxp.at[b, tid], ibuf)
        for r0, nr in ((0, FROWS), (FROWS, FROWS), (2 * FROWS, ROWS_TILE - 2 * FROWS)):
            npts = nr * 128
            if r0 + nr < ROWS_TILE:
                pltpu.sync_copy(
                    feats.at[pl.ds(pbase + r0 * 128, npts), pl.ds(c0, CH)],
                    fbuf.at[pl.ds(0, npts)])
            else:
                tail = PT_TILE - r0 * 128
                pltpu.sync_copy(
                    feats.at[pl.ds(pbase + r0 * 128, tail), pl.ds(c0, CH)],
                    fbuf.at[pl.ds(0, tail)])

                @pl.when(tid == NTILE - 1)
                def _():
                    pltpu.sync_copy(
                        feats.at[pl.ds(pbase + PT_TILE, PT_LAST - PT_TILE),
                                 pl.ds(c0, CH)],
                        fbuf.at[pl.ds(tail, PT_LAST - PT_TILE)])

            for j in range(nr):
                pltpu.sync_copy(fbuf.at[pl.ds(j * 128, 128)],
                                acc.at[ibuf.at[r0 + j]], add=True)
        plsc.subcore_barrier()

        dbase = tid * DPT
        pltpu.sync_copy(acc.at[pl.ds(dbase, DPT)],
                        out.at[b, pl.ds(dbase, DPT), pl.ds(c0, CH)])
        plsc.subcore_barrier()


def _splat(feats, idxp):
    mesh = plsc.VectorSubcoreMesh(core_axis_name="c", subcore_axis_name="s")
    return pl.kernel(
        _splat_body,
        out_type=jax.ShapeDtypeStruct((B, NVOX, C), jnp.float32),
        mesh=mesh,
        scratch_types=[
            pltpu.VMEM((FPTS, CH), jnp.float32),
            pltpu.VMEM((ROWS_TILE, 128), jnp.int32),
            pltpu.VMEM((ZROWS, CH), jnp.float32),
            pltpu.VMEM_SHARED((ACC_ROWS, CH), jnp.float32),
        ],
        compiler_params=pltpu.CompilerParams(use_tc_tiling_on_sc=False),
    )(feats, idxp)


def kernel(cam_feats, rots, trans, intrins, post_rots, post_trans):
    ds = jnp.arange(4.0, 45.0, 1.0, dtype=jnp.float32).reshape(-1, 1, 1) * jnp.ones(
        (1, fH, fW), jnp.float32)
    xs = jnp.linspace(0.0, ogfW - 1.0, fW, dtype=jnp.float32).reshape(1, 1, fW) * jnp.ones(
        (D, fH, 1), jnp.float32)
    ys = jnp.linspace(0.0, ogfH - 1.0, fH, dtype=jnp.float32).reshape(1, fH, 1) * jnp.ones(
        (D, 1, fW), jnp.float32)
    frustum = jnp.stack((xs, ys, ds), -1)
    points = frustum[None, None] - post_trans.reshape(B, N, 1, 1, 1, 3)
    inv_pr = jnp.linalg.inv(post_rots).reshape(B, N, 1, 1, 1, 3, 3)
    points = jnp.matmul(inv_pr, points[..., None])
    points = jnp.concatenate(
        (points[..., :2, :] * points[..., 2:3, :], points[..., 2:3, :]), axis=-2)
    combine = jnp.matmul(rots, jnp.linalg.inv(intrins))
    points = jnp.matmul(combine.reshape(B, N, 1, 1, 1, 3, 3), points).squeeze(-1)
    points = points + trans.reshape(B, N, 1, 1, 1, 3)

    idx = _point_indices(points.reshape(NCAM, D, HW, 3))
    idx = idx.reshape(B, P_BATCH)

    idxp = jnp.full((B, NTILE, PT_PAD), NVOX, jnp.int32)
    idxp = idxp.at[:, :NTILE - 1, :PT_TILE].set(
        idx[:, :(NTILE - 1) * PT_TILE].reshape(B, NTILE - 1, PT_TILE))
    idxp = idxp.at[:, NTILE - 1, :PT_LAST].set(idx[:, (NTILE - 1) * PT_TILE:])
    idxp = idxp.reshape(B, NTILE, ROWS_TILE, 128)

    feats = cam_feats.reshape(NPRIME, C)
    vox = _splat(feats, idxp)
    return vox.reshape(B, NX0, NX1, C).transpose(0, 3, 1, 2)


# speedup vs baseline: 1.7156x; 1.0232x over previous
import functools

import jax
import jax.numpy as jnp
from jax import lax
from jax.experimental import pallas as pl
from jax.experimental.pallas import tpu as pltpu
from jax.experimental.pallas import tpu_sc as plsc

B, N, D, fH, fW, C = 4, 6, 41, 8, 22, 64
ogfH, ogfW = 128, 352
NX0, NX1, NZ = 200, 200, 1
HW = fH * fW
P_CAM = D * HW
NCAM = B * N
NPRIME = NCAM * P_CAM
P_BATCH = NPRIME // B
NVOX = NX0 * NX1

NCORE, NTILE = 2, 16
CH = C // NCORE
PT_TILE = 2704
PT_LAST = P_BATCH - (NTILE - 1) * PT_TILE
ROWS_TILE = 22
PT_PAD = ROWS_TILE * 128
FROWS = 5
FPTS = FROWS * 128
CHUNKS = ((0, 5), (5, 5), (10, 5), (15, 5), (20, 2))
ACC_ROWS = 40016
ZROWS = 128
DPT = NVOX // NTILE


def _idx_body(px_ref, py_ref, pz_ref, idx_ref):
    gx = ((px_ref[...] - (-50.0)) / 0.5).astype(jnp.int32)
    gy = ((py_ref[...] - (-50.0)) / 0.5).astype(jnp.int32)
    gz = ((pz_ref[...] - (-10.0)) / 20.0).astype(jnp.int32)
    kept = ((gx >= 0) & (gx < NX0) & (gy >= 0) & (gy < NX1)
            & (gz >= 0) & (gz < NZ))
    idx_ref[...] = jnp.where(kept, gx * NX1 + gy, NVOX)


def _point_indices(points):
    px = points[..., 0]
    py = points[..., 1]
    pz = points[..., 2]
    return pl.pallas_call(
        _idx_body,
        out_shape=jax.ShapeDtypeStruct((NCAM, D, HW), jnp.int32),
    )(px, py, pz)


def _splat_body(feats, idxp, out, fbufA, fbufB, ibufA, ibufB, zbuf, acc,
                semLA, semLB, semSA, semSB, semI, semD, semZ):
    cid = lax.axis_index("c")
    tid = lax.axis_index("s")
    c0 = cid * CH
    fbufs = (fbufA, fbufB)
    ibufs = (ibufA, ibufB)
    semL = (semLA, semLB)
    semS = (semSA, semSB)

    zeros16 = jnp.zeros((16,), jnp.float32)

    @pl.loop(0, ZROWS)
    def _(r):
        zbuf[r, pl.ds(0, 16)] = zeros16
        zbuf[r, pl.ds(16, 16)] = zeros16

    dbase = tid * DPT

    def fire_zeros():
        descs = []
        off = 0
        while off < DPT:
            n = min(ZROWS, DPT - off)
            descs.append(pltpu.async_copy(
                zbuf.at[pl.ds(0, n)], acc.at[pl.ds(dbase + off, n)], semZ))
            off += n
        return descs

    def fire_load(b, c, g):
        r0, nr = CHUNKS[c]
        buf = fbufs[g % 2]
        pbase = b * P_BATCH + tid * PT_TILE
        if r0 + nr < ROWS_TILE:
            return [pltpu.async_copy(
                feats.at[pl.ds(pbase + r0 * 128, nr * 128), pl.ds(c0, CH)],
                buf.at[pl.ds(0, nr * 128)], semL[g % 2])]
        tail = PT_TILE - r0 * 128
        d = [pltpu.async_copy(
            feats.at[pl.ds(pbase + r0 * 128, tail), pl.ds(c0, CH)],
            buf.at[pl.ds(0, tail)], semL[g % 2])]

        @pl.when(tid == NTILE - 1)
        def _():
            pltpu.sync_copy(
                feats.at[pl.ds(pbase + PT_TILE, PT_LAST - PT_TILE),
                         pl.ds(c0, CH)],
                buf.at[pl.ds(tail, PT_LAST - PT_TILE)])

        return d

    zd = fire_zeros()
    id0 = pltpu.async_copy(idxp.at[0, tid], ibufA, semI)
    loads = {0: fire_load(0, 0, 0)}
    for d in zd:
        d.wait()
    plsc.subcore_barrier()

    pend_scat = {0: [], 1: []}
    id_next = id0
    NCHUNK = len(CHUNKS)
    for b in range(B):
        ib = ibufs[b % 2]
        id_next.wait()
        if b + 1 < B:
            id_next = pltpu.async_copy(idxp.at[b + 1, tid], ibufs[(b + 1) % 2],
                                       semI)
        for c in range(NCHUNK):
            g = b * NCHUNK + c
            for d in loads.pop(g):
                d.wait()
            if g + 1 < B * NCHUNK:
                nb, nc = divmod(g + 1, NCHUNK)
                for d in pend_scat[(g + 1) % 2]:
                    d.wait()
                pend_scat[(g + 1) % 2] = []
                loads[g + 1] = fire_load(nb, nc, g + 1)
            r0, nr = CHUNKS[c]
            buf = fbufs[g % 2]
            for j in range(nr):
                pend_scat[g % 2].append(pltpu.async_copy(
                    buf.at[pl.ds(j * 128, 128)], acc.at[ib.at[r0 + j]],
                    semS[g % 2], add=True))
        for p in (0, 1):
            for d in pend_scat[p]:
                d.wait()
            pend_scat[p] = []
        plsc.subcore_barrier()
        pltpu.async_copy(acc.at[pl.ds(dbase, DPT)],
                         out.at[b, pl.ds(dbase, DPT), pl.ds(c0, CH)],
                         semD).wait()
        if b + 1 < B:
            for d in fire_zeros():
                d.wait()
        plsc.subcore_barrier()


def _splat(feats, idxp):
    mesh = plsc.VectorSubcoreMesh(core_axis_name="c", subcore_axis_name="s")
    return pl.kernel(
        _splat_body,
        out_type=jax.ShapeDtypeStruct((B, NVOX, C), jnp.float32),
        mesh=mesh,
        scratch_types=[
            pltpu.VMEM((FPTS, CH), jnp.float32),
            pltpu.VMEM((FPTS, CH), jnp.float32),
            pltpu.VMEM((ROWS_TILE, 128), jnp.int32),
            pltpu.VMEM((ROWS_TILE, 128), jnp.int32),
            pltpu.VMEM((ZROWS, CH), jnp.float32),
            pltpu.VMEM_SHARED((ACC_ROWS, CH), jnp.float32),
            pltpu.SemaphoreType.DMA,
            pltpu.SemaphoreType.DMA,
            pltpu.SemaphoreType.DMA,
            pltpu.SemaphoreType.DMA,
            pltpu.SemaphoreType.DMA,
            pltpu.SemaphoreType.DMA,
            pltpu.SemaphoreType.DMA,
        ],
        compiler_params=pltpu.CompilerParams(use_tc_tiling_on_sc=False),
    )(feats, idxp)


def kernel(cam_feats, rots, trans, intrins, post_rots, post_trans):
    ds = jnp.arange(4.0, 45.0, 1.0, dtype=jnp.float32).reshape(-1, 1, 1) * jnp.ones(
        (1, fH, fW), jnp.float32)
    xs = jnp.linspace(0.0, ogfW - 1.0, fW, dtype=jnp.float32).reshape(1, 1, fW) * jnp.ones(
        (D, fH, 1), jnp.float32)
    ys = jnp.linspace(0.0, ogfH - 1.0, fH, dtype=jnp.float32).reshape(1, fH, 1) * jnp.ones(
        (D, 1, fW), jnp.float32)
    frustum = jnp.stack((xs, ys, ds), -1)
    points = frustum[None, None] - post_trans.reshape(B, N, 1, 1, 1, 3)
    inv_pr = jnp.linalg.inv(post_rots).reshape(B, N, 1, 1, 1, 3, 3)
    points = jnp.matmul(inv_pr, points[..., None])
    points = jnp.concatenate(
        (points[..., :2, :] * points[..., 2:3, :], points[..., 2:3, :]), axis=-2)
    combine = jnp.matmul(rots, jnp.linalg.inv(intrins))
    points = jnp.matmul(combine.reshape(B, N, 1, 1, 1, 3, 3), points).squeeze(-1)
    points = points + trans.reshape(B, N, 1, 1, 1, 3)

    idx = _point_indices(points.reshape(NCAM, D, HW, 3))
    idx = idx.reshape(B, P_BATCH)

    idxp = jnp.full((B, NTILE, PT_PAD), NVOX, jnp.int32)
    idxp = idxp.at[:, :NTILE - 1, :PT_TILE].set(
        idx[:, :(NTILE - 1) * PT_TILE].reshape(B, NTILE - 1, PT_TILE))
    idxp = idxp.at[:, NTILE - 1, :PT_LAST].set(idx[:, (NTILE - 1) * PT_TILE:])
    idxp = idxp.reshape(B, NTILE, ROWS_TILE, 128)

    feats = cam_feats.reshape(NPRIME, C)
    vox = _splat(feats, idxp)
    return vox.reshape(B, NX0, NX1, C).transpose(0, 3, 1, 2)


# speedup vs baseline: 2.2768x; 1.3271x over previous
import functools

import jax
import jax.numpy as jnp
from jax import lax
from jax.experimental import pallas as pl
from jax.experimental.pallas import tpu as pltpu
from jax.experimental.pallas import tpu_sc as plsc

B, N, D, fH, fW, C = 4, 6, 41, 8, 22, 64
ogfH, ogfW = 128, 352
NX0, NX1, NZ = 200, 200, 1
HW = fH * fW
P_CAM = D * HW
NCAM = B * N
NPRIME = NCAM * P_CAM
P_BATCH = NPRIME // B
NVOX = NX0 * NX1

NCORE, NTILE = 2, 16
CH = C // NCORE
PT_TILE = 2704
PT_LAST = P_BATCH - (NTILE - 1) * PT_TILE
ROWS_TILE = 22
PT_PAD = ROWS_TILE * 128
FROWS = 5
FPTS = FROWS * 128
CHUNKS = ((0, 5), (5, 5), (10, 5), (15, 5), (20, 2))
ACC_ROWS = 40016
ZROWS = 128
DPT = NVOX // NTILE


def _idx_body(px_ref, py_ref, pz_ref, idx_ref):
    gx = ((px_ref[...] - (-50.0)) / 0.5).astype(jnp.int32)
    gy = ((py_ref[...] - (-50.0)) / 0.5).astype(jnp.int32)
    gz = ((pz_ref[...] - (-10.0)) / 20.0).astype(jnp.int32)
    kept = ((gx >= 0) & (gx < NX0) & (gy >= 0) & (gy < NX1)
            & (gz >= 0) & (gz < NZ))
    idx_ref[...] = jnp.where(kept, gx * NX1 + gy, NVOX)


def _point_indices(points):
    px = points[..., 0]
    py = points[..., 1]
    pz = points[..., 2]
    return pl.pallas_call(
        _idx_body,
        out_shape=jax.ShapeDtypeStruct((NCAM, D, HW), jnp.int32),
    )(px, py, pz)


def _splat_body(feats, idxp, out, fbuf, ibuf, zbuf, acc, semD, semZ):
    cid = lax.axis_index("c")
    tid = lax.axis_index("s")
    c0 = cid * CH

    zeros16 = jnp.zeros((16,), jnp.float32)

    @pl.loop(0, ZROWS)
    def _(r):
        zbuf[r, pl.ds(0, 16)] = zeros16
        zbuf[r, pl.ds(16, 16)] = zeros16

    dbase = tid * DPT

    def zero_acc():
        descs = []
        off = 0
        while off < DPT:
            n = min(ZROWS, DPT - off)
            descs.append(pltpu.async_copy(
                zbuf.at[pl.ds(0, n)], acc.at[pl.ds(dbase + off, n)], semZ))
            off += n
        for d in descs:
            d.wait()

    def row_kept_count(j):
        cntv = jnp.zeros((16,), jnp.int32)
        for k in range(8):
            iv = ibuf[j, pl.ds(k * 16, 16)]
            cntv = cntv + jnp.where(iv != NVOX, 1, 0).astype(jnp.int32)
        return jnp.sum(cntv)

    zero_acc()
    plsc.subcore_barrier()

    for b in range(B):
        pbase = b * P_BATCH + tid * PT_TILE
        pltpu.sync_copy(idxp.at[b, tid], ibuf)

        @pl.loop(0, ROWS_TILE - 1)
        def _(j):
            @pl.when(row_kept_count(j) > 0)
            def _():
                pltpu.sync_copy(
                    feats.at[pl.ds(pbase + j * 128, 128), pl.ds(c0, CH)],
                    fbuf)
                pltpu.sync_copy(fbuf, acc.at[ibuf.at[j]], add=True)

        jt = ROWS_TILE - 1

        @pl.when(row_kept_count(jt) > 0)
        def _():
            tail = PT_TILE - jt * 128
            pltpu.sync_copy(
                feats.at[pl.ds(pbase + jt * 128, tail), pl.ds(c0, CH)],
                fbuf.at[pl.ds(0, tail)])

            @pl.when(tid == NTILE - 1)
            def _():
                pltpu.sync_copy(
                    feats.at[pl.ds(pbase + PT_TILE, PT_LAST - PT_TILE),
                             pl.ds(c0, CH)],
                    fbuf.at[pl.ds(tail, PT_LAST - PT_TILE)])

            pltpu.sync_copy(fbuf, acc.at[ibuf.at[jt]], add=True)

        plsc.subcore_barrier()
        pltpu.async_copy(acc.at[pl.ds(dbase, DPT)],
                         out.at[b, pl.ds(dbase, DPT), pl.ds(c0, CH)],
                         semD).wait()
        if b + 1 < B:
            zero_acc()
        plsc.subcore_barrier()


def _splat(feats, idxp):
    mesh = plsc.VectorSubcoreMesh(core_axis_name="c", subcore_axis_name="s")
    return pl.kernel(
        _splat_body,
        out_type=jax.ShapeDtypeStruct((B, NVOX, C), jnp.float32),
        mesh=mesh,
        scratch_types=[
            pltpu.VMEM((128, CH), jnp.float32),
            pltpu.VMEM((ROWS_TILE, 128), jnp.int32),
            pltpu.VMEM((ZROWS, CH), jnp.float32),
            pltpu.VMEM_SHARED((ACC_ROWS, CH), jnp.float32),
            pltpu.SemaphoreType.DMA,
            pltpu.SemaphoreType.DMA,
        ],
        compiler_params=pltpu.CompilerParams(use_tc_tiling_on_sc=False,
                                             needs_layout_passes=False),
    )(feats, idxp)


def kernel(cam_feats, rots, trans, intrins, post_rots, post_trans):
    ds = jnp.arange(4.0, 45.0, 1.0, dtype=jnp.float32).reshape(-1, 1, 1) * jnp.ones(
        (1, fH, fW), jnp.float32)
    xs = jnp.linspace(0.0, ogfW - 1.0, fW, dtype=jnp.float32).reshape(1, 1, fW) * jnp.ones(
        (D, fH, 1), jnp.float32)
    ys = jnp.linspace(0.0, ogfH - 1.0, fH, dtype=jnp.float32).reshape(1, fH, 1) * jnp.ones(
        (D, 1, fW), jnp.float32)
    frustum = jnp.stack((xs, ys, ds), -1)
    points = frustum[None, None] - post_trans.reshape(B, N, 1, 1, 1, 3)
    inv_pr = jnp.linalg.inv(post_rots).reshape(B, N, 1, 1, 1, 3, 3)
    points = jnp.matmul(inv_pr, points[..., None])
    points = jnp.concatenate(
        (points[..., :2, :] * points[..., 2:3, :], points[..., 2:3, :]), axis=-2)
    combine = jnp.matmul(rots, jnp.linalg.inv(intrins))
    points = jnp.matmul(combine.reshape(B, N, 1, 1, 1, 3, 3), points).squeeze(-1)
    points = points + trans.reshape(B, N, 1, 1, 1, 3)

    idx = _point_indices(points.reshape(NCAM, D, HW, 3))
    idx = idx.reshape(B, P_BATCH)

    idxp = jnp.full((B, NTILE, PT_PAD), NVOX, jnp.int32)
    idxp = idxp.at[:, :NTILE - 1, :PT_TILE].set(
        idx[:, :(NTILE - 1) * PT_TILE].reshape(B, NTILE - 1, PT_TILE))
    idxp = idxp.at[:, NTILE - 1, :PT_LAST].set(idx[:, (NTILE - 1) * PT_TILE:])
    idxp = idxp.reshape(B, NTILE, ROWS_TILE, 128)

    feats = cam_feats.reshape(NPRIME, C)
    vox = _splat(feats, idxp)
    return vox.reshape(B, NX0, NX1, C).transpose(0, 3, 1, 2)
